# Initial kernel scaffold; baseline (speedup 1.0000x reference)
import jax
import jax.numpy as jnp
from jax.experimental import pallas as pl


def kernel(byte_seq, bank_2, bank_4, bank_8, blade_bank_16, blade_bank_32, scale_weights):
    raise NotImplementedError("write your pallas kernel here")


# speedup vs baseline: 1.5497x; 1.5497x over previous
import functools

import jax
import jax.numpy as jnp
from jax import lax
from jax.experimental import pallas as pl
from jax.experimental.pallas import tpu as pltpu
from jax.experimental.pallas import tpu_sc as plsc

D = 8
SLOTS_2 = 65536
GLOBAL_SLOTS = 1000000
PER_BLADE_SLOTS = 100000
BATCH = 4
SEQ = 8192
NTOK = BATCH * SEQ
PADW = SEQ + 64

NW = 32
TOK_W = NTOK // NW
CH = 128
NCH = TOK_W // CH

_FNV_INIT = 2166136261
_FNV_PRIME = 16777619


def _hash_body(pad_ref, a2_ref, a4_ref, a8_ref, a16_ref, a32_ref):
    x = pad_ref[...].astype(jnp.uint32)

    def b(k):
        return x[:, k:k + SEQ]

    prime = jnp.uint32(_FNV_PRIME)
    h32 = jnp.full((BATCH, SEQ), _FNV_INIT, dtype=jnp.uint32)
    for k in range(0, 16):
        h32 = (h32 ^ b(k)) * prime
    h16 = jnp.full((BATCH, SEQ), _FNV_INIT, dtype=jnp.uint32)
    for k in range(16, 24):
        h32 = (h32 ^ b(k)) * prime
        h16 = (h16 ^ b(k)) * prime
    h8 = jnp.full((BATCH, SEQ), _FNV_INIT, dtype=jnp.uint32)
    for k in range(24, 32):
        h32 = (h32 ^ b(k)) * prime
        h16 = (h16 ^ b(k)) * prime
        h8 = (h8 ^ b(k)) * prime

    poly4 = (b(28) * jnp.uint32(1 << 24) + b(29) * jnp.uint32(1 << 16)
             + b(30) * jnp.uint32(1 << 8) + b(31))
    poly2 = b(30) * jnp.uint32(1 << 8) + b(31)

    a2_ref[...] = poly2.astype(jnp.int32)
    a4_ref[...] = (poly4 % jnp.uint32(GLOBAL_SLOTS)).astype(jnp.int32)
    a8_ref[...] = (h8 % jnp.uint32(GLOBAL_SLOTS)).astype(jnp.int32)
    a16_ref[...] = (h16 % jnp.uint32(PER_BLADE_SLOTS)).astype(jnp.int32)
    a32_ref[...] = (h32 % jnp.uint32(PER_BLADE_SLOTS)).astype(jnp.int32)


def _compute_addrs(padded):
    outs = [jax.ShapeDtypeStruct((BATCH, SEQ), jnp.int32)] * 5
    return pl.pallas_call(_hash_body, out_shape=outs)(padded)


_MEAN_COLS = 32000


def _mean_body(bank_ref, out_ref):
    out_ref[...] = jnp.sum(bank_ref[...], axis=0, keepdims=True) * 0.125


def _blade_mean(bank):
    flat = bank.reshape(8, PER_BLADE_SLOTS * D)
    grid = (PER_BLADE_SLOTS * D) // _MEAN_COLS
    out = pl.pallas_call(
        _mean_body,
        grid=(grid,),
        in_specs=[pl.BlockSpec((8, _MEAN_COLS), lambda i: (0, i))],
        out_specs=pl.BlockSpec((1, _MEAN_COLS), lambda i: (0, i)),
        out_shape=jax.ShapeDtypeStruct((1, PER_BLADE_SLOTS * D), jnp.float32),
    )(flat)
    return out.reshape(PER_BLADE_SLOTS, D)


def _sc_gather_body(a2, a4, a8, a16, a32, t2, t4, t8, m16, m32,
                    o2, o4, o8, o16, o32, idx_v, rows_v, sem):
    wid = lax.axis_index("s") * 2 + lax.axis_index("c")
    base = wid * TOK_W
    for a, tbl, o in ((a2, t2, o2), (a4, t4, o4), (a8, t8, o8),
                      (a16, m16, o16), (a32, m32, o32)):
        pltpu.sync_copy(a.at[wid], idx_v)
        cps = []
        for c in range(NCH):
            cps.append(pltpu.async_copy(
                tbl.at[idx_v.at[c]], rows_v.at[pl.ds(c * CH, CH)], sem))
        for cp in cps:
            cp.wait()
        pltpu.sync_copy(rows_v, o.at[pl.ds(base, TOK_W)])


@functools.partial(
    pl.kernel,
    out_type=[jax.ShapeDtypeStruct((NTOK, D), jnp.float32)] * 5,
    mesh=plsc.VectorSubcoreMesh(core_axis_name="c", subcore_axis_name="s"),
    scratch_types=[
        pltpu.VMEM((NCH, CH), jnp.int32),
        pltpu.VMEM((TOK_W, D), jnp.float32),
        pltpu.SemaphoreType.DMA,
    ],
    compiler_params=pltpu.CompilerParams(use_tc_tiling_on_sc=False),
)
def _sc_gather(*refs):
    _sc_gather_body(*refs)


_CMB_ROWS = 256


def _combine_body(w_ref, r2, r4, r8, r16, r32, out_ref):
    out_ref[...] = (w_ref[0] * r2[...] + w_ref[1] * r4[...]
                    + w_ref[2] * r8[...] + w_ref[3] * r16[...]
                    + w_ref[4] * r32[...])


def _combine(w, rs):
    flat = [r.reshape(NTOK * D // 128, 128) for r in rs]
    grid = (NTOK * D // 128) // _CMB_ROWS
    blk = pl.BlockSpec((_CMB_ROWS, 128), lambda i: (i, 0))
    out = pl.pallas_call(
        _combine_body,
        grid=(grid,),
        in_specs=[pl.BlockSpec(memory_space=pltpu.SMEM)] + [blk] * 5,
        out_specs=blk,
        out_shape=jax.ShapeDtypeStruct((NTOK * D // 128, 128), jnp.float32),
    )(w, *flat)
    return out.reshape(BATCH, SEQ, D)


def kernel(byte_seq, bank_2, bank_4, bank_8, blade_bank_16, blade_bank_32,
           scale_weights):
    padded = jnp.pad(byte_seq, ((0, 0), (31, PADW - SEQ - 31)))
    addrs = _compute_addrs(padded)
    addrs = [a.reshape(NW, NCH, CH) for a in addrs]
    m16 = _blade_mean(blade_bank_16)
    m32 = _blade_mean(blade_bank_32)
    rs = _sc_gather(*addrs, bank_2, bank_4, bank_8, m16, m32)
    return _combine(scale_weights, rs)


# speedup vs baseline: 15.0761x; 9.7286x over previous
import functools

import jax
import jax.numpy as jnp
from jax import lax
from jax.experimental import pallas as pl
from jax.experimental.pallas import tpu as pltpu
from jax.experimental.pallas import tpu_sc as plsc

D = 8
SLOTS_2 = 65536
GLOBAL_SLOTS = 1000000
PER_BLADE_SLOTS = 100000
BATCH = 4
SEQ = 8192
NTOK = BATCH * SEQ
PADW = SEQ + 64

NW = 32
TOK_W = NTOK // NW
CH = 128
NCH = TOK_W // CH

RB_2 = SLOTS_2 // 128
STRIDE_2 = RB_2 * 128
RB_G = 7936
STRIDE_G = RB_G * 128
GRID_G = 123
RB_B = 832
STRIDE_B = RB_B * 128
GRID_B = 13

_FNV_INIT = 2166136261
_FNV_PRIME = 16777619


def _hash_body(pad_ref, i2_ref, i4_ref, i8_ref, i16_ref, i32_ref):
    x = pad_ref[...].astype(jnp.uint32)

    def b(k):
        return x[:, k:k + SEQ]

    prime = jnp.uint32(_FNV_PRIME)
    h32 = jnp.full((BATCH, SEQ), _FNV_INIT, dtype=jnp.uint32)
    for k in range(0, 16):
        h32 = (h32 ^ b(k)) * prime
    h16 = jnp.full((BATCH, SEQ), _FNV_INIT, dtype=jnp.uint32)
    for k in range(16, 24):
        h32 = (h32 ^ b(k)) * prime
        h16 = (h16 ^ b(k)) * prime
    h8 = jnp.full((BATCH, SEQ), _FNV_INIT, dtype=jnp.uint32)
    for k in range(24, 32):
        h32 = (h32 ^ b(k)) * prime
        h16 = (h16 ^ b(k)) * prime
        h8 = (h8 ^ b(k)) * prime

    poly4 = (b(28) * jnp.uint32(1 << 24) + b(29) * jnp.uint32(1 << 16)
             + b(30) * jnp.uint32(1 << 8) + b(31))
    poly2 = b(30) * jnp.uint32(1 << 8) + b(31)

    addrs = (
        (poly2, STRIDE_2, i2_ref),
        (poly4 % jnp.uint32(GLOBAL_SLOTS), STRIDE_G, i4_ref),
        (h8 % jnp.uint32(GLOBAL_SLOTS), STRIDE_G, i8_ref),
        (h16 % jnp.uint32(PER_BLADE_SLOTS), STRIDE_B, i16_ref),
        (h32 % jnp.uint32(PER_BLADE_SLOTS), STRIDE_B, i32_ref),
    )
    doff = lax.broadcasted_iota(jnp.int32, (1, 1, D, 1), 2)
    for a, stride, ref in addrs:
        a = a.astype(jnp.int32).reshape(BATCH, SEQ // TOK_W, 1, TOK_W)
        e = a + doff * jnp.int32(stride)
        ref[...] = e.reshape(NW, D, NCH, CH).reshape(NW, D * NCH, CH)


def _compute_indices(padded):
    outs = [jax.ShapeDtypeStruct((NW, D * NCH, CH), jnp.int32)] * 5
    return pl.pallas_call(_hash_body, out_shape=outs)(padded)


def _repack_body(in_ref, out_ref):
    out_ref[...] = in_ref[...].reshape(D, 64, 128)


def _repack(table_t, rb, grid_j):
    return pl.pallas_call(
        _repack_body,
        grid=(grid_j,),
        in_specs=[pl.BlockSpec((D, 64 * 128), lambda j: (0, j))],
        out_specs=pl.BlockSpec((D, 64, 128), lambda j: (0, j, 0)),
        out_shape=jax.ShapeDtypeStruct((D, rb, 128), jnp.float32),
    )(table_t)


def _mean_body(bank_ref, out_ref):
    out_ref[...] = (jnp.sum(bank_ref[...], axis=0) * 0.125).reshape(D, 64, 128)


def _blade_mean(bank_t):
    return pl.pallas_call(
        _mean_body,
        grid=(GRID_B,),
        in_specs=[pl.BlockSpec((8, D, 64 * 128), lambda j: (0, 0, j))],
        out_specs=pl.BlockSpec((D, 64, 128), lambda j: (0, j, 0)),
        out_shape=jax.ShapeDtypeStruct((D, RB_B, 128), jnp.float32),
    )(bank_t)


def _sc_gather_body(i2, i4, i8, i16, i32, t2, t4, t8, m16, m32,
                    o2, o4, o8, o16, o32, idx_v, rows_v, sem):
    wid = lax.axis_index("s") * 2 + lax.axis_index("c")
    for ii, tbl, o in ((i2, t2, o2), (i4, t4, o4), (i8, t8, o8),
                       (i16, m16, o16), (i32, m32, o32)):
        pltpu.sync_copy(ii.at[wid], idx_v)
        cps = []
        for r in range(D * NCH):
            cps.append(pltpu.async_copy(
                tbl.at[idx_v.at[r]], rows_v.at[r], sem))
        for cp in cps:
            cp.wait()
        pltpu.sync_copy(rows_v, o.at[wid])


@functools.partial(
    pl.kernel,
    out_type=[jax.ShapeDtypeStruct((NW, D * NCH, CH), jnp.float32)] * 5,
    mesh=plsc.VectorSubcoreMesh(core_axis_name="c", subcore_axis_name="s"),
    scratch_types=[
        pltpu.VMEM((D * NCH, CH), jnp.int32),
        pltpu.VMEM((D * NCH, CH), jnp.float32),
        pltpu.SemaphoreType.DMA,
    ],
    compiler_params=pltpu.CompilerParams(use_tc_tiling_on_sc=False),
)
def _sc_gather(*refs):
    _sc_gather_body(*refs)


def _combine_body(w_ref, r2, r4, r8, r16, r32, out_ref):
    acc = (w_ref[0] * r2[...] + w_ref[1] * r4[...] + w_ref[2] * r8[...]
           + w_ref[3] * r16[...] + w_ref[4] * r32[...])
    out_ref[...] = acc.reshape(1, D, TOK_W)


def _combine(w, rs):
    rblk = pl.BlockSpec((1, D * NCH, CH), lambda i: (i, 0, 0))
    out = pl.pallas_call(
        _combine_body,
        grid=(NW,),
        in_specs=[pl.BlockSpec(memory_space=pltpu.SMEM)] + [rblk] * 5,
        out_specs=pl.BlockSpec((1, D, TOK_W), lambda i: (i // 8, 0, i % 8)),
        out_shape=jax.ShapeDtypeStruct((BATCH, D, SEQ), jnp.float32),
    )(w, *rs)
    return out.transpose(0, 2, 1)


def kernel(byte_seq, bank_2, bank_4, bank_8, blade_bank_16, blade_bank_32,
           scale_weights):
    padded = jnp.pad(byte_seq, ((0, 0), (31, PADW - SEQ - 31)))
    idxs = _compute_indices(padded)
    t2 = _repack(bank_2.T, RB_2, RB_2 // 64).reshape(-1)
    t4 = _repack(bank_4.T, RB_G, GRID_G).reshape(-1)
    t8 = _repack(bank_8.T, RB_G, GRID_G).reshape(-1)
    m16 = _blade_mean(blade_bank_16.transpose(0, 2, 1)).reshape(-1)
    m32 = _blade_mean(blade_bank_32.transpose(0, 2, 1)).reshape(-1)
    rs = _sc_gather(*idxs, t2, t4, t8, m16, m32)
    return _combine(scale_weights, rs)


# speedup vs baseline: 25.3909x; 1.6842x over previous
import functools

import jax
import jax.numpy as jnp
from jax import lax
from jax.experimental import pallas as pl
from jax.experimental.pallas import tpu as pltpu
from jax.experimental.pallas import tpu_sc as plsc

D = 8
SLOTS_2 = 65536
GLOBAL_SLOTS = 1000000
PER_BLADE_SLOTS = 100000
BATCH = 4
SEQ = 8192
NTOK = BATCH * SEQ
PADW = SEQ + 64

NW = 32
TOK_W = NTOK // NW
CH = 128
NCH = TOK_W // CH

RB_2 = SLOTS_2 // 128
STRIDE_2 = RB_2 * 128
SLB_G = 131072
RB_G = 8192
STRIDE_G = RB_G * 128
GRID_G = 8
SLB_B = 16384
RB_B = 896
STRIDE_B = RB_B * 128
GRID_B = 7

_FNV_INIT = 2166136261
_FNV_PRIME = 16777619


def _hash_body(pad_ref, i2_ref, i4_ref, i8_ref, i16_ref, i32_ref):
    x = pad_ref[...].astype(jnp.uint32)

    def b(k):
        return x[:, k:k + SEQ]

    prime = jnp.uint32(_FNV_PRIME)
    h32 = jnp.full((BATCH, SEQ), _FNV_INIT, dtype=jnp.uint32)
    for k in range(0, 16):
        h32 = (h32 ^ b(k)) * prime
    h16 = jnp.full((BATCH, SEQ), _FNV_INIT, dtype=jnp.uint32)
    for k in range(16, 24):
        h32 = (h32 ^ b(k)) * prime
        h16 = (h16 ^ b(k)) * prime
    h8 = jnp.full((BATCH, SEQ), _FNV_INIT, dtype=jnp.uint32)
    for k in range(24, 32):
        h32 = (h32 ^ b(k)) * prime
        h16 = (h16 ^ b(k)) * prime
        h8 = (h8 ^ b(k)) * prime

    poly4 = (b(28) * jnp.uint32(1 << 24) + b(29) * jnp.uint32(1 << 16)
             + b(30) * jnp.uint32(1 << 8) + b(31))
    poly2 = b(30) * jnp.uint32(1 << 8) + b(31)

    addrs = (
        (poly2, STRIDE_2, i2_ref),
        (poly4 % jnp.uint32(GLOBAL_SLOTS), STRIDE_G, i4_ref),
        (h8 % jnp.uint32(GLOBAL_SLOTS), STRIDE_G, i8_ref),
        (h16 % jnp.uint32(PER_BLADE_SLOTS), STRIDE_B, i16_ref),
        (h32 % jnp.uint32(PER_BLADE_SLOTS), STRIDE_B, i32_ref),
    )
    doff = lax.broadcasted_iota(jnp.int32, (1, 1, D, 1), 2)
    for a, stride, ref in addrs:
        a = a.astype(jnp.int32).reshape(BATCH, SEQ // TOK_W, 1, TOK_W)
        e = a + doff * jnp.int32(stride)
        ref[...] = e.reshape(NW, D, NCH, CH).reshape(NW, D * NCH, CH)


def _compute_indices(padded):
    outs = [jax.ShapeDtypeStruct((NW, D * NCH, CH), jnp.int32)] * 5
    return pl.pallas_call(_hash_body, out_shape=outs)(padded)


def _make_repack_body(slb):
    def _repack_body(in_ref, out_ref):
        out_ref[...] = in_ref[...].reshape(D, slb // 128, 128)
    return _repack_body


def _repack(table_t, rb, grid_j, slb):
    return pl.pallas_call(
        _make_repack_body(slb),
        grid=(grid_j,),
        in_specs=[pl.BlockSpec((D, slb), lambda j: (0, j))],
        out_specs=pl.BlockSpec((D, slb // 128, 128), lambda j: (0, j, 0)),
        out_shape=jax.ShapeDtypeStruct((D, rb, 128), jnp.float32),
    )(table_t)


def _mean_body(bank_ref, out_ref):
    out_ref[...] = (jnp.sum(bank_ref[...], axis=0) * 0.125).reshape(
        D, SLB_B // 128, 128)


def _blade_mean(bank_t):
    return pl.pallas_call(
        _mean_body,
        grid=(GRID_B,),
        in_specs=[pl.BlockSpec((8, D, SLB_B), lambda j: (0, 0, j))],
        out_specs=pl.BlockSpec((D, SLB_B // 128, 128), lambda j: (0, j, 0)),
        out_shape=jax.ShapeDtypeStruct((D, RB_B, 128), jnp.float32),
    )(bank_t)


def _sc_gather_body(i2, i4, i8, i16, i32, t2, t4, t8, m16, m32,
                    o2, o4, o8, o16, o32, idx_v, rows_v, sem):
    wid = lax.axis_index("s") * 2 + lax.axis_index("c")
    for ii, tbl, o in ((i2, t2, o2), (i4, t4, o4), (i8, t8, o8),
                       (i16, m16, o16), (i32, m32, o32)):
        pltpu.sync_copy(ii.at[wid], idx_v)
        cps = []
        for r in range(D * NCH):
            cps.append(pltpu.async_copy(
                tbl.at[idx_v.at[r]], rows_v.at[r], sem))
        for cp in cps:
            cp.wait()
        pltpu.sync_copy(rows_v, o.at[wid])


@functools.partial(
    pl.kernel,
    out_type=[jax.ShapeDtypeStruct((NW, D * NCH, CH), jnp.float32)] * 5,
    mesh=plsc.VectorSubcoreMesh(core_axis_name="c", subcore_axis_name="s"),
    scratch_types=[
        pltpu.VMEM((D * NCH, CH), jnp.int32),
        pltpu.VMEM((D * NCH, CH), jnp.float32),
        pltpu.SemaphoreType.DMA,
    ],
    compiler_params=pltpu.CompilerParams(use_tc_tiling_on_sc=False),
)
def _sc_gather(*refs):
    _sc_gather_body(*refs)


def _combine_body(w_ref, r2, r4, r8, r16, r32, out_ref):
    acc = (w_ref[0] * r2[...] + w_ref[1] * r4[...] + w_ref[2] * r8[...]
           + w_ref[3] * r16[...] + w_ref[4] * r32[...])
    out_ref[...] = acc.reshape(1, D, TOK_W)


def _combine(w, rs):
    rblk = pl.BlockSpec((1, D * NCH, CH), lambda i: (i, 0, 0))
    out = pl.pallas_call(
        _combine_body,
        grid=(NW,),
        in_specs=[pl.BlockSpec(memory_space=pltpu.SMEM)] + [rblk] * 5,
        out_specs=pl.BlockSpec((1, D, TOK_W), lambda i: (i // 8, 0, i % 8)),
        out_shape=jax.ShapeDtypeStruct((BATCH, D, SEQ), jnp.float32),
    )(w, *rs)
    return out.transpose(0, 2, 1)


def kernel(byte_seq, bank_2, bank_4, bank_8, blade_bank_16, blade_bank_32,
           scale_weights):
    padded = jnp.pad(byte_seq, ((0, 0), (31, PADW - SEQ - 31)))
    idxs = _compute_indices(padded)
    t2 = _repack(bank_2.T, RB_2, 1, SLOTS_2).reshape(-1)
    t4 = _repack(bank_4.T, RB_G, GRID_G, SLB_G).reshape(-1)
    t8 = _repack(bank_8.T, RB_G, GRID_G, SLB_G).reshape(-1)
    m16 = _blade_mean(blade_bank_16.transpose(0, 2, 1)).reshape(-1)
    m32 = _blade_mean(blade_bank_32.transpose(0, 2, 1)).reshape(-1)
    rs = _sc_gather(*idxs, t2, t4, t8, m16, m32)
    return _combine(scale_weights, rs)
